# Initial kernel scaffold; baseline (speedup 1.0000x reference)
import jax
import jax.numpy as jnp
from jax.experimental import pallas as pl


def kernel(x, edge_index, W1, b1, W2, b2):
    raise NotImplementedError("write your pallas kernel here")


# speedup vs baseline: 28.1882x; 28.1882x over previous
import functools

import jax
import jax.numpy as jnp
from jax import lax
from jax.experimental import pallas as pl
from jax.experimental.pallas import tpu as pltpu
from jax.experimental.pallas import tpu_sc as plsc

N = 10000
E = 320000
D_IN = 128
D = 16
D_OUT = 7

NTILE = 16
NSC = 2
NW = NTILE * NSC

NPAD = 10240
RPT = NPAD // NTILE
EPAD = 327680
BATCH = 128
NB_ALL = EPAD // BATCH
NB_DEG = NB_ALL // NTILE
NB = NB_ALL // NW

_MESH = plsc.VectorSubcoreMesh(core_axis_name="c", subcore_axis_name="s")


@functools.partial(
    pl.kernel,
    out_type=jax.ShapeDtypeStruct((NPAD,), jnp.float32),
    mesh=_MESH,
    scratch_types=[
        pltpu.VMEM((NB_DEG, BATCH), jnp.int32),
        pltpu.VMEM((BATCH,), jnp.float32),
        pltpu.VMEM((RPT,), jnp.float32),
        pltpu.VMEM_SHARED((NPAD,), jnp.float32),
    ],
)
def _sc_degree(dst_hbm, deg_hbm, dst_v, ones_v, zb_v, deg_s):
    c = lax.axis_index("c")
    s = lax.axis_index("s")

    def fill_ones(i, carry):
        ones_v[pl.ds(i * 16, 16)] = jnp.full((16,), 1.0, jnp.float32)
        return carry

    lax.fori_loop(0, BATCH // 16, fill_ones, 0)

    def fill_zero(i, carry):
        zb_v[pl.ds(i * 16, 16)] = jnp.zeros((16,), jnp.float32)
        return carry

    lax.fori_loop(0, RPT // 16, fill_zero, 0)

    pltpu.sync_copy(zb_v, deg_s.at[pl.ds(s * RPT, RPT)])
    pltpu.sync_copy(dst_hbm.at[pl.ds(s * NB_DEG, NB_DEG)], dst_v)
    plsc.subcore_barrier()

    def step(g, carry):
        pltpu.sync_copy(ones_v, deg_s.at[dst_v.at[g]], add=True)
        return carry

    lax.fori_loop(0, NB_DEG, step, 0)
    plsc.subcore_barrier()

    @pl.when(c == 0)
    def _():
        pltpu.sync_copy(deg_s.at[pl.ds(s * RPT, RPT)],
                        deg_hbm.at[pl.ds(s * RPT, RPT)])


@functools.partial(
    pl.kernel,
    out_type=jax.ShapeDtypeStruct((NSC, NPAD, D), jnp.float32),
    mesh=_MESH,
    scratch_types=[
        pltpu.VMEM((NB, BATCH), jnp.int32),
        pltpu.VMEM((NB, BATCH), jnp.int32),
        pltpu.VMEM((BATCH, D), jnp.float32),
        pltpu.VMEM((RPT, D), jnp.float32),
        pltpu.VMEM_SHARED((NPAD, D), jnp.float32),
        pltpu.SemaphoreType.DMA,
    ],
    compiler_params=pltpu.CompilerParams(use_tc_tiling_on_sc=False),
)
def _sc_prop(g_hbm, src_hbm, dst_hbm, p_hbm, src_v, dst_v, rows_v, zb_v,
             acc_s, sem):
    c = lax.axis_index("c")
    s = lax.axis_index("s")
    wid = c * NTILE + s

    def fill_zero(i, carry):
        zb_v[i] = jnp.zeros((16,), jnp.float32)
        return carry

    lax.fori_loop(0, RPT, fill_zero, 0, unroll=4)

    pltpu.sync_copy(zb_v, acc_s.at[pl.ds(s * RPT, RPT)])
    pltpu.sync_copy(src_hbm.at[pl.ds(wid * NB, NB)], src_v)
    pltpu.sync_copy(dst_hbm.at[pl.ds(wid * NB, NB)], dst_v)
    plsc.subcore_barrier()

    def step(g, carry):
        pltpu.async_copy(g_hbm.at[src_v.at[g]], rows_v, sem).wait()
        pltpu.sync_copy(rows_v, acc_s.at[dst_v.at[g]], add=True)
        return carry

    lax.fori_loop(0, NB, step, 0)
    plsc.subcore_barrier()

    pltpu.sync_copy(acc_s.at[pl.ds(s * RPT, RPT)],
                    p_hbm.at[c, pl.ds(s * RPT, RPT)])


RBLK = 1024


def _tc1_body(x_ref, w_ref, deg_ref, g_ref, dis_ref):
    d = deg_ref[...] + 1.0
    dis = lax.rsqrt(d)
    h = jnp.dot(x_ref[...], w_ref[...], preferred_element_type=jnp.float32,
                precision=lax.Precision.HIGHEST)
    g_ref[...] = dis * h
    dis_ref[...] = dis


_tc1 = pl.pallas_call(
    _tc1_body,
    grid=(NPAD // RBLK,),
    in_specs=[
        pl.BlockSpec((RBLK, D_IN), lambda i: (i, 0)),
        pl.BlockSpec((D_IN, D), lambda i: (0, 0)),
        pl.BlockSpec((RBLK, 1), lambda i: (i, 0)),
    ],
    out_specs=[
        pl.BlockSpec((RBLK, D), lambda i: (i, 0)),
        pl.BlockSpec((RBLK, 1), lambda i: (i, 0)),
    ],
    out_shape=[
        jax.ShapeDtypeStruct((NPAD, D), jnp.float32),
        jax.ShapeDtypeStruct((NPAD, 1), jnp.float32),
    ],
)


def _tc2_body(p_ref, g1_ref, dis_ref, b1_ref, w2_ref, g2_ref):
    s = (p_ref[0] + p_ref[1] + g1_ref[...]) * dis_ref[...] + b1_ref[...]
    h = jnp.maximum(s, 0.0)
    g2_ref[...] = dis_ref[...] * jnp.dot(
        h, w2_ref[...], preferred_element_type=jnp.float32,
        precision=lax.Precision.HIGHEST)


_tc2 = pl.pallas_call(
    _tc2_body,
    grid=(NPAD // RBLK,),
    in_specs=[
        pl.BlockSpec((NSC, RBLK, D), lambda i: (0, i, 0)),
        pl.BlockSpec((RBLK, D), lambda i: (i, 0)),
        pl.BlockSpec((RBLK, 1), lambda i: (i, 0)),
        pl.BlockSpec((1, D), lambda i: (0, 0)),
        pl.BlockSpec((D, D), lambda i: (0, 0)),
    ],
    out_specs=pl.BlockSpec((RBLK, D), lambda i: (i, 0)),
    out_shape=jax.ShapeDtypeStruct((NPAD, D), jnp.float32),
)


def _tc3_body(q_ref, g2_ref, dis_ref, b2_ref, o_ref):
    o_ref[...] = ((q_ref[0] + q_ref[1] + g2_ref[...]) * dis_ref[...]
                  + b2_ref[...])


_tc3 = pl.pallas_call(
    _tc3_body,
    grid=(NPAD // RBLK,),
    in_specs=[
        pl.BlockSpec((NSC, RBLK, D), lambda i: (0, i, 0)),
        pl.BlockSpec((RBLK, D), lambda i: (i, 0)),
        pl.BlockSpec((RBLK, 1), lambda i: (i, 0)),
        pl.BlockSpec((1, D), lambda i: (0, 0)),
    ],
    out_specs=pl.BlockSpec((RBLK, D), lambda i: (i, 0)),
    out_shape=jax.ShapeDtypeStruct((NPAD, D), jnp.float32),
)


def kernel(x, edge_index, W1, b1, W2, b2):
    x_pad = jnp.pad(x, ((0, NPAD - N), (0, 0)))
    pad_e = EPAD - E
    src_p = jnp.concatenate(
        [edge_index[0], jnp.zeros((pad_e,), jnp.int32)]).reshape(NB_ALL, BATCH)
    dst_p = jnp.concatenate(
        [edge_index[1], jnp.full((pad_e,), N, jnp.int32)]).reshape(NB_ALL, BATCH)

    deg = _sc_degree(dst_p)
    g1, dis2d = _tc1(x_pad, W1, deg.reshape(NPAD, 1))
    p = _sc_prop(g1, src_p, dst_p)

    w2p = jnp.pad(W2, ((0, 0), (0, D - D_OUT)))
    b1r = b1.reshape(1, D)
    b2r = jnp.pad(b2, (0, D - D_OUT)).reshape(1, D)

    g2 = _tc2(p, g1, dis2d, b1r, w2p)
    q = _sc_prop(g2, src_p, dst_p)
    out_pad = _tc3(q, g2, dis2d, b2r)
    return out_pad[:N, :D_OUT]


# speedup vs baseline: 35.8519x; 1.2719x over previous
import functools

import jax
import jax.numpy as jnp
from jax import lax
from jax.experimental import pallas as pl
from jax.experimental.pallas import tpu as pltpu
from jax.experimental.pallas import tpu_sc as plsc

N = 10000
E = 320000
D_IN = 128
D = 16
D_OUT = 7

NTILE = 16
NSC = 2
NW = NTILE * NSC

NPAD = 10240
RPT = NPAD // NTILE
EPAD = 327680
BATCH = 128
NB_ALL = EPAD // BATCH
NB_DEG = NB_ALL // NTILE
NB = NB_ALL // NW

_MESH = plsc.VectorSubcoreMesh(core_axis_name="c", subcore_axis_name="s")


@functools.partial(
    pl.kernel,
    out_type=jax.ShapeDtypeStruct((NPAD,), jnp.float32),
    mesh=_MESH,
    scratch_types=[
        pltpu.VMEM((NB_DEG, BATCH), jnp.int32),
        pltpu.VMEM((BATCH,), jnp.float32),
        pltpu.VMEM((RPT,), jnp.float32),
        pltpu.VMEM_SHARED((NPAD,), jnp.float32),
    ],
)
def _sc_degree(dst_hbm, deg_hbm, dst_v, ones_v, zb_v, deg_s):
    c = lax.axis_index("c")
    s = lax.axis_index("s")

    def fill_ones(i, carry):
        ones_v[pl.ds(i * 16, 16)] = jnp.full((16,), 1.0, jnp.float32)
        return carry

    lax.fori_loop(0, BATCH // 16, fill_ones, 0)

    def fill_zero(i, carry):
        zb_v[pl.ds(i * 16, 16)] = jnp.zeros((16,), jnp.float32)
        return carry

    lax.fori_loop(0, RPT // 16, fill_zero, 0)

    pltpu.sync_copy(zb_v, deg_s.at[pl.ds(s * RPT, RPT)])
    pltpu.sync_copy(dst_hbm.at[pl.ds(s * NB_DEG, NB_DEG)], dst_v)
    plsc.subcore_barrier()

    def step(g, carry):
        pltpu.sync_copy(ones_v, deg_s.at[dst_v.at[g]], add=True)
        return carry

    lax.fori_loop(0, NB_DEG, step, 0)
    plsc.subcore_barrier()

    @pl.when(c == 0)
    def _():
        pltpu.sync_copy(deg_s.at[pl.ds(s * RPT, RPT)],
                        deg_hbm.at[pl.ds(s * RPT, RPT)])


NCHUNK = 4
KH = NB // NCHUNK
CROWS = KH * BATCH


@functools.partial(
    pl.kernel,
    out_type=jax.ShapeDtypeStruct((NSC, NPAD, D), jnp.float32),
    mesh=_MESH,
    scratch_types=[
        pltpu.VMEM((NCHUNK, CROWS), jnp.int32),
        pltpu.VMEM((NCHUNK, CROWS), jnp.int32),
        pltpu.VMEM((CROWS, D), jnp.float32),
        pltpu.VMEM((CROWS, D), jnp.float32),
        pltpu.VMEM((RPT, D), jnp.float32),
        pltpu.VMEM_SHARED((NPAD, D), jnp.float32),
        pltpu.SemaphoreType.DMA,
        pltpu.SemaphoreType.DMA,
    ],
    compiler_params=pltpu.CompilerParams(use_tc_tiling_on_sc=False),
)
def _sc_prop(g_hbm, src_hbm, dst_hbm, p_hbm, src_v, dst_v, rows0_v, rows1_v,
             zb_v, acc_s, sem0, sem1):
    c = lax.axis_index("c")
    s = lax.axis_index("s")
    wid = c * NTILE + s

    cps = pltpu.async_copy(src_hbm.at[wid], src_v, sem0)
    cpd = pltpu.async_copy(dst_hbm.at[wid], dst_v, sem1)

    def fill_zero(i, carry):
        zb_v[i] = jnp.zeros((16,), jnp.float32)
        return carry

    lax.fori_loop(0, RPT, fill_zero, 0, unroll=4)

    cps.wait()
    cpd.wait()
    pltpu.sync_copy(zb_v, acc_s.at[pl.ds(s * RPT, RPT)])
    plsc.subcore_barrier()

    rows = (rows0_v, rows1_v)
    sems = (sem0, sem1)
    pltpu.async_copy(g_hbm.at[src_v.at[0]], rows0_v, sem0)
    for ci in range(NCHUNK):
        if ci + 1 < NCHUNK:
            pltpu.async_copy(g_hbm.at[src_v.at[ci + 1]],
                             rows[(ci + 1) % 2], sems[(ci + 1) % 2])
        pltpu.make_async_copy(g_hbm.at[src_v.at[ci]],
                              rows[ci % 2], sems[ci % 2]).wait()
        pltpu.sync_copy(rows[ci % 2], acc_s.at[dst_v.at[ci]], add=True)
    plsc.subcore_barrier()

    pltpu.sync_copy(acc_s.at[pl.ds(s * RPT, RPT)],
                    p_hbm.at[c, pl.ds(s * RPT, RPT)])


RBLK = 1024


def _tc1_body(x_ref, w_ref, deg_ref, g_ref, dis_ref):
    d = deg_ref[...] + 1.0
    dis = lax.rsqrt(d)
    h = jnp.dot(x_ref[...], w_ref[...], preferred_element_type=jnp.float32,
                precision=lax.Precision.HIGHEST)
    g_ref[...] = dis * h
    dis_ref[...] = dis


_tc1 = pl.pallas_call(
    _tc1_body,
    grid=(NPAD // RBLK,),
    in_specs=[
        pl.BlockSpec((RBLK, D_IN), lambda i: (i, 0)),
        pl.BlockSpec((D_IN, D), lambda i: (0, 0)),
        pl.BlockSpec((RBLK, 1), lambda i: (i, 0)),
    ],
    out_specs=[
        pl.BlockSpec((RBLK, D), lambda i: (i, 0)),
        pl.BlockSpec((RBLK, 1), lambda i: (i, 0)),
    ],
    out_shape=[
        jax.ShapeDtypeStruct((NPAD, D), jnp.float32),
        jax.ShapeDtypeStruct((NPAD, 1), jnp.float32),
    ],
)


def _tc2_body(p_ref, g1_ref, dis_ref, b1_ref, w2_ref, g2_ref):
    s = (p_ref[0] + p_ref[1] + g1_ref[...]) * dis_ref[...] + b1_ref[...]
    h = jnp.maximum(s, 0.0)
    g2_ref[...] = dis_ref[...] * jnp.dot(
        h, w2_ref[...], preferred_element_type=jnp.float32,
        precision=lax.Precision.HIGHEST)


_tc2 = pl.pallas_call(
    _tc2_body,
    grid=(NPAD // RBLK,),
    in_specs=[
        pl.BlockSpec((NSC, RBLK, D), lambda i: (0, i, 0)),
        pl.BlockSpec((RBLK, D), lambda i: (i, 0)),
        pl.BlockSpec((RBLK, 1), lambda i: (i, 0)),
        pl.BlockSpec((1, D), lambda i: (0, 0)),
        pl.BlockSpec((D, D), lambda i: (0, 0)),
    ],
    out_specs=pl.BlockSpec((RBLK, D), lambda i: (i, 0)),
    out_shape=jax.ShapeDtypeStruct((NPAD, D), jnp.float32),
)


def _tc3_body(q_ref, g2_ref, dis_ref, b2_ref, o_ref):
    o_ref[...] = ((q_ref[0] + q_ref[1] + g2_ref[...]) * dis_ref[...]
                  + b2_ref[...])


_tc3 = pl.pallas_call(
    _tc3_body,
    grid=(NPAD // RBLK,),
    in_specs=[
        pl.BlockSpec((NSC, RBLK, D), lambda i: (0, i, 0)),
        pl.BlockSpec((RBLK, D), lambda i: (i, 0)),
        pl.BlockSpec((RBLK, 1), lambda i: (i, 0)),
        pl.BlockSpec((1, D), lambda i: (0, 0)),
    ],
    out_specs=pl.BlockSpec((RBLK, D), lambda i: (i, 0)),
    out_shape=jax.ShapeDtypeStruct((NPAD, D), jnp.float32),
)


def kernel(x, edge_index, W1, b1, W2, b2):
    x_pad = jnp.pad(x, ((0, NPAD - N), (0, 0)))
    pad_e = EPAD - E
    src_p = jnp.concatenate(
        [edge_index[0], jnp.zeros((pad_e,), jnp.int32)]).reshape(NB_ALL, BATCH)
    dst_p = jnp.concatenate(
        [edge_index[1], jnp.full((pad_e,), N, jnp.int32)]).reshape(NB_ALL, BATCH)

    src4 = src_p.reshape(NW, NCHUNK, CROWS)
    dst4 = dst_p.reshape(NW, NCHUNK, CROWS)

    deg = _sc_degree(dst_p)
    g1, dis2d = _tc1(x_pad, W1, deg.reshape(NPAD, 1))
    p = _sc_prop(g1, src4, dst4)

    w2p = jnp.pad(W2, ((0, 0), (0, D - D_OUT)))
    b1r = b1.reshape(1, D)
    b2r = jnp.pad(b2, (0, D - D_OUT)).reshape(1, D)

    g2 = _tc2(p, g1, dis2d, b1r, w2p)
    q = _sc_prop(g2, src4, dst4)
    out_pad = _tc3(q, g2, dis2d, b2r)
    return out_pad[:N, :D_OUT]


# speedup vs baseline: 54.7061x; 1.5259x over previous
import functools

import jax
import jax.numpy as jnp
from jax import lax
from jax.experimental import pallas as pl
from jax.experimental.pallas import tpu as pltpu
from jax.experimental.pallas import tpu_sc as plsc

N = 10000
E = 320000
D_IN = 128
D = 16
D2 = 8
D_OUT = 7

NTILE = 16
NSC = 2
NW = NTILE * NSC

NPAD = 10240
RPT = NPAD // NTILE
EPAD = 327680
BATCH = 128
NB_ALL = EPAD // BATCH
NB_DEG = NB_ALL // NTILE
NB = NB_ALL // NW

_MESH = plsc.VectorSubcoreMesh(core_axis_name="c", subcore_axis_name="s")


@functools.partial(
    pl.kernel,
    out_type=jax.ShapeDtypeStruct((NSC, NPAD), jnp.float32),
    mesh=_MESH,
    scratch_types=[
        pltpu.VMEM((NB, BATCH), jnp.int32),
        pltpu.VMEM((BATCH,), jnp.float32),
        pltpu.VMEM_SHARED((NPAD,), jnp.float32),
        pltpu.SemaphoreType.DMA,
        pltpu.SemaphoreType.DMA,
    ],
)
def _sc_degree(dst_hbm, ones_hbm, z1_hbm, deg_hbm, dst_v, ones_v, deg_s,
               sem, ssem):
    c = lax.axis_index("c")
    s = lax.axis_index("s")
    wid = c * NTILE + s

    cpd = pltpu.async_copy(dst_hbm.at[wid], dst_v, sem)
    pltpu.sync_copy(ones_hbm, ones_v)
    pltpu.sync_copy(z1_hbm.at[pl.ds(s * RPT, RPT)],
                    deg_s.at[pl.ds(s * RPT, RPT)])
    cpd.wait()
    plsc.subcore_barrier()

    def issue(g, carry):
        pltpu.async_copy(ones_v, deg_s.at[dst_v.at[g]], ssem, add=True)
        return carry

    lax.fori_loop(0, NB, issue, 0)

    def drain(g, carry):
        pltpu.make_async_copy(ones_v, deg_s.at[dst_v.at[0]], ssem).wait()
        return carry

    lax.fori_loop(0, NB, drain, 0)
    plsc.subcore_barrier()

    pltpu.sync_copy(deg_s.at[pl.ds(s * RPT, RPT)],
                    deg_hbm.at[c, pl.ds(s * RPT, RPT)])


NCHUNK = 4
KH = NB // NCHUNK
CROWS = KH * BATCH


def _make_prop(d):
    @functools.partial(
        pl.kernel,
        out_type=jax.ShapeDtypeStruct((NSC, NPAD, d), jnp.float32),
        mesh=_MESH,
        scratch_types=[
            pltpu.VMEM((NCHUNK, CROWS), jnp.int32),
            pltpu.VMEM((NCHUNK, CROWS), jnp.int32),
            pltpu.VMEM((CROWS, d), jnp.float32),
            pltpu.VMEM((CROWS, d), jnp.float32),
            pltpu.VMEM_SHARED((NPAD, d), jnp.float32),
            pltpu.VMEM_SHARED((NPAD, d), jnp.float32),
            pltpu.SemaphoreType.DMA,
            pltpu.SemaphoreType.DMA,
            pltpu.SemaphoreType.DMA,
        ],
        compiler_params=pltpu.CompilerParams(use_tc_tiling_on_sc=False),
    )
    def _sc_prop(g_hbm, src_hbm, dst_hbm, z2_hbm, p_hbm, src_v, dst_v,
                 rows0_v, rows1_v, acc_s, g_s, sem0, sem1, semg):
        c = lax.axis_index("c")
        s = lax.axis_index("s")
        wid = c * NTILE + s

        cps = pltpu.async_copy(src_hbm.at[wid], src_v, sem0)
        cpd = pltpu.async_copy(dst_hbm.at[wid], dst_v, sem1)
        cpg = pltpu.async_copy(g_hbm.at[pl.ds(s * RPT, RPT)],
                               g_s.at[pl.ds(s * RPT, RPT)], semg)
        pltpu.sync_copy(z2_hbm.at[pl.ds(s * RPT, RPT)],
                        acc_s.at[pl.ds(s * RPT, RPT)])
        cps.wait()
        cpd.wait()
        cpg.wait()
        plsc.subcore_barrier()

        rows = (rows0_v, rows1_v)
        sems = (sem0, sem1)
        pltpu.async_copy(g_s.at[src_v.at[0]], rows0_v, sem0)
        for ci in range(NCHUNK):
            if ci + 1 < NCHUNK:
                pltpu.async_copy(g_s.at[src_v.at[ci + 1]],
                                 rows[(ci + 1) % 2], sems[(ci + 1) % 2])
            pltpu.make_async_copy(g_s.at[src_v.at[ci]],
                                  rows[ci % 2], sems[ci % 2]).wait()
            pltpu.sync_copy(rows[ci % 2], acc_s.at[dst_v.at[ci]], add=True)
        plsc.subcore_barrier()

        pltpu.sync_copy(acc_s.at[pl.ds(s * RPT, RPT)],
                        p_hbm.at[c, pl.ds(s * RPT, RPT)])

    return _sc_prop


_sc_prop1 = _make_prop(D)
_sc_prop2 = _make_prop(D2)


RBLK = 1024


def _tc1_body(x_ref, w_ref, deg_ref, g_ref, dis_ref):
    d = deg_ref[0] + deg_ref[1] + 1.0
    dis = lax.rsqrt(d)
    h = jnp.dot(x_ref[...], w_ref[...], preferred_element_type=jnp.float32,
                precision=lax.Precision.HIGHEST)
    g_ref[...] = dis * h
    dis_ref[...] = dis


_tc1 = pl.pallas_call(
    _tc1_body,
    grid=(NPAD // RBLK,),
    in_specs=[
        pl.BlockSpec((RBLK, D_IN), lambda i: (i, 0)),
        pl.BlockSpec((D_IN, D), lambda i: (0, 0)),
        pl.BlockSpec((NSC, RBLK, 1), lambda i: (0, i, 0)),
    ],
    out_specs=[
        pl.BlockSpec((RBLK, D), lambda i: (i, 0)),
        pl.BlockSpec((RBLK, 1), lambda i: (i, 0)),
    ],
    out_shape=[
        jax.ShapeDtypeStruct((NPAD, D), jnp.float32),
        jax.ShapeDtypeStruct((NPAD, 1), jnp.float32),
    ],
)


def _tc2_body(p_ref, g1_ref, dis_ref, b1_ref, w2_ref, g2_ref):
    s = (p_ref[0] + p_ref[1] + g1_ref[...]) * dis_ref[...] + b1_ref[...]
    h = jnp.maximum(s, 0.0)
    g2_ref[...] = dis_ref[...] * jnp.dot(
        h, w2_ref[...], preferred_element_type=jnp.float32,
        precision=lax.Precision.HIGHEST)


_tc2 = pl.pallas_call(
    _tc2_body,
    grid=(NPAD // RBLK,),
    in_specs=[
        pl.BlockSpec((NSC, RBLK, D), lambda i: (0, i, 0)),
        pl.BlockSpec((RBLK, D), lambda i: (i, 0)),
        pl.BlockSpec((RBLK, 1), lambda i: (i, 0)),
        pl.BlockSpec((1, D), lambda i: (0, 0)),
        pl.BlockSpec((D, D2), lambda i: (0, 0)),
    ],
    out_specs=pl.BlockSpec((RBLK, D2), lambda i: (i, 0)),
    out_shape=jax.ShapeDtypeStruct((NPAD, D2), jnp.float32),
)


def _tc3_body(q_ref, g2_ref, dis_ref, b2_ref, o_ref):
    o_ref[...] = ((q_ref[0] + q_ref[1] + g2_ref[...]) * dis_ref[...]
                  + b2_ref[...])


_tc3 = pl.pallas_call(
    _tc3_body,
    grid=(NPAD // RBLK,),
    in_specs=[
        pl.BlockSpec((NSC, RBLK, D2), lambda i: (0, i, 0)),
        pl.BlockSpec((RBLK, D2), lambda i: (i, 0)),
        pl.BlockSpec((RBLK, 1), lambda i: (i, 0)),
        pl.BlockSpec((1, D2), lambda i: (0, 0)),
    ],
    out_specs=pl.BlockSpec((RBLK, D2), lambda i: (i, 0)),
    out_shape=jax.ShapeDtypeStruct((NPAD, D2), jnp.float32),
)


def kernel(x, edge_index, W1, b1, W2, b2):
    x_pad = jnp.pad(x, ((0, NPAD - N), (0, 0)))
    pad_e = EPAD - E
    src_p = jnp.concatenate(
        [edge_index[0], jnp.zeros((pad_e,), jnp.int32)])
    dst_p = jnp.concatenate(
        [edge_index[1], jnp.full((pad_e,), N, jnp.int32)])

    src4 = src_p.reshape(NW, NCHUNK, CROWS)
    dst4 = dst_p.reshape(NW, NCHUNK, CROWS)

    ones_b = jnp.ones((BATCH,), jnp.float32)
    z1 = jnp.zeros((NPAD,), jnp.float32)
    z2a = jnp.zeros((NPAD, D), jnp.float32)
    z2b = jnp.zeros((NPAD, D2), jnp.float32)

    deg = _sc_degree(dst_p.reshape(NW, NB, BATCH), ones_b, z1)
    g1, dis2d = _tc1(x_pad, W1, deg.reshape(NSC, NPAD, 1))
    p = _sc_prop1(g1, src4, dst4, z2a)

    w2p = jnp.pad(W2, ((0, 0), (0, D2 - D_OUT)))
    b1r = b1.reshape(1, D)
    b2r = jnp.pad(b2, (0, D2 - D_OUT)).reshape(1, D2)

    g2 = _tc2(p, g1, dis2d, b1r, w2p)
    q = _sc_prop2(g2, src4, dst4, z2b)
    out_pad = _tc3(q, g2, dis2d, b2r)
    return out_pad[:N, :D_OUT]


# speedup vs baseline: 57.6954x; 1.0546x over previous
import functools

import jax
import jax.numpy as jnp
from jax import lax
from jax.experimental import pallas as pl
from jax.experimental.pallas import tpu as pltpu
from jax.experimental.pallas import tpu_sc as plsc

N = 10000
E = 320000
D_IN = 128
D = 16
D2 = 8
D_OUT = 7

NTILE = 16
NSC = 2
NW = NTILE * NSC

NPAD = 10240
RPT = NPAD // NTILE
EPAD = 327680
BATCH = 128
NB_ALL = EPAD // BATCH
NB_DEG = NB_ALL // NTILE
NB = NB_ALL // NW

_MESH = plsc.VectorSubcoreMesh(core_axis_name="c", subcore_axis_name="s")


@functools.partial(
    pl.kernel,
    out_type=jax.ShapeDtypeStruct((NSC, NPAD), jnp.float32),
    mesh=_MESH,
    scratch_types=[
        pltpu.VMEM((NB, BATCH), jnp.int32),
        pltpu.VMEM((BATCH,), jnp.float32),
        pltpu.VMEM_SHARED((NPAD,), jnp.float32),
        pltpu.SemaphoreType.DMA,
        pltpu.SemaphoreType.DMA,
    ],
)
def _sc_degree(edge_hbm, ones_hbm, z1_hbm, deg_hbm, dst_v, ones_v, deg_s,
               sem, ssem):
    c = lax.axis_index("c")
    s = lax.axis_index("s")
    wid = c * NTILE + s

    cpd = pltpu.async_copy(edge_hbm.at[1, wid], dst_v, sem)
    pltpu.sync_copy(ones_hbm, ones_v)
    pltpu.sync_copy(z1_hbm.at[pl.ds(s * RPT, RPT)],
                    deg_s.at[pl.ds(s * RPT, RPT)])
    cpd.wait()
    plsc.subcore_barrier()

    def issue(g, carry):
        pltpu.async_copy(ones_v, deg_s.at[dst_v.at[g]], ssem, add=True)
        return carry

    lax.fori_loop(0, NB, issue, 0)

    def drain(g, carry):
        pltpu.make_async_copy(ones_v, deg_s.at[dst_v.at[0]], ssem).wait()
        return carry

    lax.fori_loop(0, NB, drain, 0)
    plsc.subcore_barrier()

    pltpu.sync_copy(deg_s.at[pl.ds(s * RPT, RPT)],
                    deg_hbm.at[c, pl.ds(s * RPT, RPT)])


NCHUNK = 4
KH = NB // NCHUNK
CROWS = KH * BATCH


def _make_prop(d):
    @functools.partial(
        pl.kernel,
        out_type=jax.ShapeDtypeStruct((NSC, NPAD, d), jnp.float32),
        mesh=_MESH,
        scratch_types=[
            pltpu.VMEM((NCHUNK, CROWS), jnp.int32),
            pltpu.VMEM((NCHUNK, CROWS), jnp.int32),
            pltpu.VMEM((CROWS, d), jnp.float32),
            pltpu.VMEM((CROWS, d), jnp.float32),
            pltpu.VMEM_SHARED((NPAD, d), jnp.float32),
            pltpu.VMEM_SHARED((NPAD, d), jnp.float32),
            pltpu.SemaphoreType.DMA,
            pltpu.SemaphoreType.DMA,
            pltpu.SemaphoreType.DMA,
        ],
        compiler_params=pltpu.CompilerParams(use_tc_tiling_on_sc=False),
    )
    def _sc_prop(g_hbm, edge_hbm, z2_hbm, p_hbm, src_v, dst_v,
                 rows0_v, rows1_v, acc_s, g_s, sem0, sem1, semg):
        c = lax.axis_index("c")
        s = lax.axis_index("s")
        wid = c * NTILE + s

        cps = pltpu.async_copy(edge_hbm.at[0, wid], src_v, sem0)
        cpd = pltpu.async_copy(edge_hbm.at[1, wid], dst_v, sem1)
        cpg = pltpu.async_copy(g_hbm.at[pl.ds(s * RPT, RPT)],
                               g_s.at[pl.ds(s * RPT, RPT)], semg)
        pltpu.sync_copy(z2_hbm.at[pl.ds(s * RPT, RPT)],
                        acc_s.at[pl.ds(s * RPT, RPT)])
        cps.wait()
        cpd.wait()
        cpg.wait()
        plsc.subcore_barrier()

        rows = (rows0_v, rows1_v)
        sems = (sem0, sem1)
        pltpu.async_copy(g_s.at[src_v.at[0]], rows0_v, sem0)
        for ci in range(NCHUNK):
            if ci + 1 < NCHUNK:
                pltpu.async_copy(g_s.at[src_v.at[ci + 1]],
                                 rows[(ci + 1) % 2], sems[(ci + 1) % 2])
            pltpu.make_async_copy(g_s.at[src_v.at[ci]],
                                  rows[ci % 2], sems[ci % 2]).wait()
            pltpu.sync_copy(rows[ci % 2], acc_s.at[dst_v.at[ci]], add=True)
        plsc.subcore_barrier()

        pltpu.sync_copy(acc_s.at[pl.ds(s * RPT, RPT)],
                        p_hbm.at[c, pl.ds(s * RPT, RPT)])

    return _sc_prop


_sc_prop1 = _make_prop(D)
_sc_prop2 = _make_prop(D2)


RBLK = 1024


def _tc1_body(x_ref, w_ref, deg_ref, g_ref, dis_ref):
    d = deg_ref[0] + deg_ref[1] + 1.0
    dis = lax.rsqrt(d)
    h = jnp.dot(x_ref[...], w_ref[...], preferred_element_type=jnp.float32,
                precision=lax.Precision.HIGHEST)
    g_ref[...] = dis * h
    dis_ref[...] = dis


_tc1 = pl.pallas_call(
    _tc1_body,
    grid=(NPAD // RBLK,),
    in_specs=[
        pl.BlockSpec((RBLK, D_IN), lambda i: (i, 0)),
        pl.BlockSpec((D_IN, D), lambda i: (0, 0)),
        pl.BlockSpec((NSC, RBLK, 1), lambda i: (0, i, 0)),
    ],
    out_specs=[
        pl.BlockSpec((RBLK, D), lambda i: (i, 0)),
        pl.BlockSpec((RBLK, 1), lambda i: (i, 0)),
    ],
    out_shape=[
        jax.ShapeDtypeStruct((NPAD, D), jnp.float32),
        jax.ShapeDtypeStruct((NPAD, 1), jnp.float32),
    ],
)


def _tc2_body(p_ref, g1_ref, dis_ref, b1_ref, w2_ref, g2_ref):
    s = (p_ref[0] + p_ref[1] + g1_ref[...]) * dis_ref[...] + b1_ref[...]
    h = jnp.maximum(s, 0.0)
    g2_ref[...] = dis_ref[...] * jnp.dot(
        h, w2_ref[...], preferred_element_type=jnp.float32,
        precision=lax.Precision.HIGHEST)


_tc2 = pl.pallas_call(
    _tc2_body,
    grid=(NPAD // RBLK,),
    in_specs=[
        pl.BlockSpec((NSC, RBLK, D), lambda i: (0, i, 0)),
        pl.BlockSpec((RBLK, D), lambda i: (i, 0)),
        pl.BlockSpec((RBLK, 1), lambda i: (i, 0)),
        pl.BlockSpec((1, D), lambda i: (0, 0)),
        pl.BlockSpec((D, D2), lambda i: (0, 0)),
    ],
    out_specs=pl.BlockSpec((RBLK, D2), lambda i: (i, 0)),
    out_shape=jax.ShapeDtypeStruct((NPAD, D2), jnp.float32),
)


def _tc3_body(q_ref, g2_ref, dis_ref, b2_ref, o_ref):
    o_ref[...] = ((q_ref[0] + q_ref[1] + g2_ref[...]) * dis_ref[...]
                  + b2_ref[...])


_tc3 = pl.pallas_call(
    _tc3_body,
    grid=(NPAD // RBLK,),
    in_specs=[
        pl.BlockSpec((NSC, RBLK, D2), lambda i: (0, i, 0)),
        pl.BlockSpec((RBLK, D2), lambda i: (i, 0)),
        pl.BlockSpec((RBLK, 1), lambda i: (i, 0)),
        pl.BlockSpec((1, D2), lambda i: (0, 0)),
    ],
    out_specs=pl.BlockSpec((RBLK, D2), lambda i: (i, 0)),
    out_shape=jax.ShapeDtypeStruct((NPAD, D2), jnp.float32),
)


def kernel(x, edge_index, W1, b1, W2, b2):
    x_pad = jnp.pad(x, ((0, NPAD - N), (0, 0)))
    ep = jnp.pad(edge_index, ((0, 0), (0, EPAD - E)), constant_values=N)
    ep_prop = ep.reshape(2, NW, NCHUNK, CROWS)
    ep_deg = ep.reshape(2, NW, NB, BATCH)

    ones_b = jnp.ones((BATCH,), jnp.float32)
    z1 = jnp.zeros((NPAD,), jnp.float32)
    z2a = jnp.zeros((NPAD, D), jnp.float32)
    z2b = jnp.zeros((NPAD, D2), jnp.float32)

    deg = _sc_degree(ep_deg, ones_b, z1)
    g1, dis2d = _tc1(x_pad, W1, deg.reshape(NSC, NPAD, 1))
    p = _sc_prop1(g1, ep_prop, z2a)

    w2p = jnp.pad(W2, ((0, 0), (0, D2 - D_OUT)))
    b1r = b1.reshape(1, D)
    b2r = jnp.pad(b2, (0, D2 - D_OUT)).reshape(1, D2)

    g2 = _tc2(p, g1, dis2d, b1r, w2p)
    q = _sc_prop2(g2, ep_prop, z2b)
    out_pad = _tc3(q, g2, dis2d, b2r)
    return out_pad[:N, :D_OUT]


# speedup vs baseline: 66.5560x; 1.1536x over previous
import functools

import jax
import jax.numpy as jnp
from jax import lax
from jax.experimental import pallas as pl
from jax.experimental.pallas import tpu as pltpu
from jax.experimental.pallas import tpu_sc as plsc

N = 10000
E = 320000
D_IN = 128
D = 16
D2 = 8
D_OUT = 7

NTILE = 16
NSC = 2
NW = NTILE * NSC

NPAD = 10240
RPT = NPAD // NTILE
EPAD = 327680
BATCH = 128
NB_ALL = EPAD // BATCH
NB_DEG = NB_ALL // NTILE
NB = NB_ALL // NW

_MESH = plsc.VectorSubcoreMesh(core_axis_name="c", subcore_axis_name="s")


@functools.partial(
    pl.kernel,
    out_type=jax.ShapeDtypeStruct((NSC, NPAD, D), jnp.float32),
    mesh=_MESH,
    scratch_types=[
        pltpu.VMEM((NB, BATCH), jnp.int32),
        pltpu.VMEM((BATCH,), jnp.float32),
        pltpu.VMEM((RPT,), jnp.float32),
        pltpu.VMEM((RPT, D), jnp.float32),
        pltpu.VMEM_SHARED((NPAD,), jnp.float32),
        pltpu.SemaphoreType.DMA,
        pltpu.SemaphoreType.DMA,
    ],
    compiler_params=pltpu.CompilerParams(needs_layout_passes=False),
)
def _sc_degree(edge_hbm, ones_hbm, z1_hbm, deg_hbm, dst_v, ones_v, dv_v, dw_v,
               deg_s, sem, ssem):
    c = lax.axis_index("c")
    s = lax.axis_index("s")
    wid = c * NTILE + s

    cpd = pltpu.async_copy(edge_hbm.at[1, wid], dst_v, sem)
    pltpu.sync_copy(ones_hbm, ones_v)
    pltpu.sync_copy(z1_hbm.at[pl.ds(s * RPT, RPT)],
                    deg_s.at[pl.ds(s * RPT, RPT)])
    cpd.wait()
    plsc.subcore_barrier()

    def issue(g, carry):
        pltpu.async_copy(ones_v, deg_s.at[dst_v.at[g]], ssem, add=True)
        return carry

    lax.fori_loop(0, NB, issue, 0)

    def drain(g, carry):
        pltpu.make_async_copy(ones_v, deg_s.at[dst_v.at[0]], ssem).wait()
        return carry

    lax.fori_loop(0, NB, drain, 0)
    plsc.subcore_barrier()

    pltpu.sync_copy(deg_s.at[pl.ds(s * RPT, RPT)], dv_v)

    def widen(j, carry):
        idx = jnp.broadcast_to(j, (16,)).astype(jnp.int32)
        dw_v[j] = plsc.load_gather(dv_v, [idx])
        return carry

    lax.fori_loop(0, RPT, widen, 0, unroll=4)
    pltpu.sync_copy(dw_v, deg_hbm.at[c, pl.ds(s * RPT, RPT)])


NCHUNK = 4
KH = NB // NCHUNK
CROWS = KH * BATCH


def _make_prop(d):
    @functools.partial(
        pl.kernel,
        out_type=jax.ShapeDtypeStruct((NSC, NPAD, d), jnp.float32),
        mesh=_MESH,
        scratch_types=[
            pltpu.VMEM((NCHUNK, CROWS), jnp.int32),
            pltpu.VMEM((NCHUNK, CROWS), jnp.int32),
            pltpu.VMEM((CROWS, d), jnp.float32),
            pltpu.VMEM((CROWS, d), jnp.float32),
            pltpu.VMEM_SHARED((NPAD, d), jnp.float32),
            pltpu.VMEM_SHARED((NPAD, d), jnp.float32),
            pltpu.SemaphoreType.DMA,
            pltpu.SemaphoreType.DMA,
            pltpu.SemaphoreType.DMA,
        ],
        compiler_params=pltpu.CompilerParams(use_tc_tiling_on_sc=False),
    )
    def _sc_prop(g_hbm, edge_hbm, z2_hbm, p_hbm, src_v, dst_v,
                 rows0_v, rows1_v, acc_s, g_s, sem0, sem1, semg):
        c = lax.axis_index("c")
        s = lax.axis_index("s")
        wid = c * NTILE + s

        cps = pltpu.async_copy(edge_hbm.at[0, wid], src_v, sem0)
        cpd = pltpu.async_copy(edge_hbm.at[1, wid], dst_v, sem1)
        cpg = pltpu.async_copy(g_hbm.at[pl.ds(s * RPT, RPT)],
                               g_s.at[pl.ds(s * RPT, RPT)], semg)
        pltpu.sync_copy(z2_hbm.at[pl.ds(s * RPT, RPT)],
                        acc_s.at[pl.ds(s * RPT, RPT)])
        cps.wait()
        cpd.wait()
        cpg.wait()
        plsc.subcore_barrier()

        rows = (rows0_v, rows1_v)
        sems = (sem0, sem1)
        pltpu.async_copy(g_s.at[src_v.at[0]], rows0_v, sem0)
        for ci in range(NCHUNK):
            if ci + 1 < NCHUNK:
                pltpu.async_copy(g_s.at[src_v.at[ci + 1]],
                                 rows[(ci + 1) % 2], sems[(ci + 1) % 2])
            pltpu.make_async_copy(g_s.at[src_v.at[ci]],
                                  rows[ci % 2], sems[ci % 2]).wait()
            pltpu.sync_copy(rows[ci % 2], acc_s.at[dst_v.at[ci]], add=True)
        plsc.subcore_barrier()

        pltpu.sync_copy(acc_s.at[pl.ds(s * RPT, RPT)],
                        p_hbm.at[c, pl.ds(s * RPT, RPT)])

    return _sc_prop


_sc_prop1 = _make_prop(D)


RBLK = 1024


PB = 128


def _tc1_body(x_ref, w_ref, deg_ref, g_ref, dis_ref):
    d = deg_ref[0] + deg_ref[1] + 1.0
    dis = lax.rsqrt(d)
    h = jnp.dot(x_ref[...], w_ref[...], preferred_element_type=jnp.float32,
                precision=lax.Precision.HIGHEST)
    g_ref[...] = dis * h
    dis_ref[...] = dis


_tc1 = pl.pallas_call(
    _tc1_body,
    grid=(NPAD // RBLK,),
    in_specs=[
        pl.BlockSpec((PB, 8 * D_IN), lambda i: (i, 0)),
        pl.BlockSpec((8 * D_IN, 128), lambda i: (0, 0)),
        pl.BlockSpec((NSC, PB, 128), lambda i: (0, i, 0)),
    ],
    out_specs=[
        pl.BlockSpec((PB, 128), lambda i: (i, 0)),
        pl.BlockSpec((PB, 128), lambda i: (i, 0)),
    ],
    out_shape=[
        jax.ShapeDtypeStruct((NPAD // 8, 128), jnp.float32),
        jax.ShapeDtypeStruct((NPAD // 8, 128), jnp.float32),
    ],
)


def _tc2_body(p_ref, g1_ref, dis_ref, b1_ref, w2_ref, g2_ref):
    s = (p_ref[0] + p_ref[1] + g1_ref[...]) * dis_ref[...] + b1_ref[...]
    h = jnp.maximum(s, 0.0)
    g2_ref[...] = dis_ref[...] * jnp.dot(
        h, w2_ref[...], preferred_element_type=jnp.float32,
        precision=lax.Precision.HIGHEST)


_tc2 = pl.pallas_call(
    _tc2_body,
    grid=(NPAD // RBLK,),
    in_specs=[
        pl.BlockSpec((NSC, PB, 128), lambda i: (0, i, 0)),
        pl.BlockSpec((PB, 128), lambda i: (i, 0)),
        pl.BlockSpec((PB, 128), lambda i: (i, 0)),
        pl.BlockSpec((1, 128), lambda i: (0, 0)),
        pl.BlockSpec((128, 128), lambda i: (0, 0)),
    ],
    out_specs=pl.BlockSpec((PB, 128), lambda i: (i, 0)),
    out_shape=jax.ShapeDtypeStruct((NPAD // 8, 128), jnp.float32),
)


def _tc3_body(q_ref, g2_ref, dis_ref, b2_ref, o_ref):
    o_ref[...] = ((q_ref[0] + q_ref[1] + g2_ref[...]) * dis_ref[...]
                  + b2_ref[...])


_tc3 = pl.pallas_call(
    _tc3_body,
    grid=(NPAD // RBLK,),
    in_specs=[
        pl.BlockSpec((NSC, PB, 128), lambda i: (0, i, 0)),
        pl.BlockSpec((PB, 128), lambda i: (i, 0)),
        pl.BlockSpec((PB, 128), lambda i: (i, 0)),
        pl.BlockSpec((1, 128), lambda i: (0, 0)),
    ],
    out_specs=pl.BlockSpec((PB, 128), lambda i: (i, 0)),
    out_shape=jax.ShapeDtypeStruct((NPAD // 8, 128), jnp.float32),
)


def kernel(x, edge_index, W1, b1, W2, b2):
    x_pad = jnp.pad(x, ((0, NPAD - N), (0, 0)))
    ep = jnp.pad(edge_index, ((0, 0), (0, EPAD - E)), constant_values=N)
    ep_prop = ep.reshape(2, NW, NCHUNK, CROWS)
    ep_deg = ep.reshape(2, NW, NB, BATCH)

    ones_b = jnp.ones((BATCH,), jnp.float32)
    z1 = jnp.zeros((NPAD,), jnp.float32)
    z2 = jnp.zeros((NPAD, D), jnp.float32)

    x_pk = x_pad.reshape(NPAD // 8, 8 * D_IN)
    w1bd = jnp.kron(jnp.eye(8, dtype=jnp.float32), W1)

    degw = _sc_degree(ep_deg, ones_b, z1)
    g1p, disp = _tc1(x_pk, w1bd, degw.reshape(NSC, NPAD // 8, 128))
    p = _sc_prop1(g1p.reshape(NPAD, D), ep_prop, z2)

    w2sq = jnp.pad(W2, ((0, 0), (0, D - D_OUT)))
    w2bd = jnp.kron(jnp.eye(8, dtype=jnp.float32), w2sq)
    b1t = jnp.tile(b1, 8).reshape(1, 128)
    b2t = jnp.tile(jnp.pad(b2, (0, D - D_OUT)), 8).reshape(1, 128)

    g2p = _tc2(p.reshape(NSC, NPAD // 8, 128), g1p, disp, b1t, w2bd)
    q = _sc_prop1(g2p.reshape(NPAD, D), ep_prop, z2)
    outp = _tc3(q.reshape(NSC, NPAD // 8, 128), g2p, disp, b2t)
    return outp.reshape(NPAD, D)[:N, :D_OUT]


# speedup vs baseline: 72.2542x; 1.0856x over previous
import functools

import jax
import jax.numpy as jnp
from jax import lax
from jax.experimental import pallas as pl
from jax.experimental.pallas import tpu as pltpu
from jax.experimental.pallas import tpu_sc as plsc

N = 10000
E = 320000
D_IN = 128
D = 16
D2 = 8
D_OUT = 7

NTILE = 16
NSC = 2
NW = NTILE * NSC

NPAD = 10240
RPT = NPAD // NTILE
EPAD = 327680
BATCH = 128
NB_ALL = EPAD // BATCH
NB_DEG = NB_ALL // NTILE
NB = NB_ALL // NW

_MESH = plsc.VectorSubcoreMesh(core_axis_name="c", subcore_axis_name="s")


@functools.partial(
    pl.kernel,
    out_type=jax.ShapeDtypeStruct((NSC, NPAD // 8, 128), jnp.float32),
    mesh=_MESH,
    scratch_types=[
        pltpu.VMEM((NB, BATCH), jnp.int32),
        pltpu.VMEM((BATCH,), jnp.float32),
        pltpu.VMEM((RPT,), jnp.float32),
        pltpu.VMEM((RPT // 8, 128), jnp.float32),
        pltpu.VMEM_SHARED((NPAD,), jnp.float32),
        pltpu.SemaphoreType.DMA,
        pltpu.SemaphoreType.DMA,
    ],
    compiler_params=pltpu.CompilerParams(needs_layout_passes=False),
)
def _sc_degree(edge_hbm, ones_hbm, z1_hbm, deg_hbm, dst_v, ones_v, dv_v, dw_v,
               deg_s, sem, ssem):
    c = lax.axis_index("c")
    s = lax.axis_index("s")
    wid = c * NTILE + s

    cpd = pltpu.async_copy(edge_hbm.at[1, wid], dst_v, sem)
    pltpu.sync_copy(ones_hbm, ones_v)
    pltpu.sync_copy(z1_hbm.at[pl.ds(s * RPT, RPT)],
                    deg_s.at[pl.ds(s * RPT, RPT)])
    cpd.wait()
    plsc.subcore_barrier()

    def issue(g, carry):
        pltpu.async_copy(ones_v, deg_s.at[dst_v.at[g]], ssem, add=True)
        return carry

    lax.fori_loop(0, NB, issue, 0)

    def drain(g, carry):
        pltpu.make_async_copy(ones_v, deg_s.at[dst_v.at[0]], ssem).wait()
        return carry

    lax.fori_loop(0, NB, drain, 0)
    plsc.subcore_barrier()

    pltpu.sync_copy(deg_s.at[pl.ds(s * RPT, RPT)], dv_v)

    def widen(j, carry):
        idx = jnp.broadcast_to(j, (16,)).astype(jnp.int32)
        dw_v[j // 8, pl.ds((j % 8) * 16, 16)] = plsc.load_gather(dv_v, [idx])
        return carry

    lax.fori_loop(0, RPT, widen, 0, unroll=4)
    pltpu.sync_copy(dw_v, deg_hbm.at[c, pl.ds(s * (RPT // 8), RPT // 8)])


NCHUNK = 4
KH = NB // NCHUNK
CROWS = KH * BATCH


def _make_prop(d):
    @functools.partial(
        pl.kernel,
        out_type=jax.ShapeDtypeStruct((NSC, NPAD, d), jnp.float32),
        mesh=_MESH,
        scratch_types=[
            pltpu.VMEM((NCHUNK, CROWS), jnp.int32),
            pltpu.VMEM((NCHUNK, CROWS), jnp.int32),
            pltpu.VMEM((CROWS, d), jnp.float32),
            pltpu.VMEM((CROWS, d), jnp.float32),
            pltpu.VMEM_SHARED((NPAD, d), jnp.float32),
            pltpu.VMEM_SHARED((NPAD, d), jnp.float32),
            pltpu.SemaphoreType.DMA,
            pltpu.SemaphoreType.DMA,
            pltpu.SemaphoreType.DMA,
        ],
        compiler_params=pltpu.CompilerParams(use_tc_tiling_on_sc=False),
    )
    def _sc_prop(g_hbm, edge_hbm, z2_hbm, p_hbm, src_v, dst_v,
                 rows0_v, rows1_v, acc_s, g_s, sem0, sem1, semg):
        c = lax.axis_index("c")
        s = lax.axis_index("s")
        wid = c * NTILE + s

        cps = pltpu.async_copy(edge_hbm.at[0, wid], src_v, sem0)
        cpd = pltpu.async_copy(edge_hbm.at[1, wid], dst_v, sem1)
        cpg = pltpu.async_copy(g_hbm.at[pl.ds(s * RPT, RPT)],
                               g_s.at[pl.ds(s * RPT, RPT)], semg)
        pltpu.sync_copy(z2_hbm.at[pl.ds(s * RPT, RPT)],
                        acc_s.at[pl.ds(s * RPT, RPT)])
        cps.wait()
        cpd.wait()
        cpg.wait()
        plsc.subcore_barrier()

        rows = (rows0_v, rows1_v)
        sems = (sem0, sem1)
        pltpu.async_copy(g_s.at[src_v.at[0]], rows0_v, sem0)
        for ci in range(NCHUNK):
            if ci + 1 < NCHUNK:
                pltpu.async_copy(g_s.at[src_v.at[ci + 1]],
                                 rows[(ci + 1) % 2], sems[(ci + 1) % 2])
            pltpu.make_async_copy(g_s.at[src_v.at[ci]],
                                  rows[ci % 2], sems[ci % 2]).wait()
            pltpu.sync_copy(rows[ci % 2], acc_s.at[dst_v.at[ci]], add=True)
        plsc.subcore_barrier()

        pltpu.sync_copy(acc_s.at[pl.ds(s * RPT, RPT)],
                        p_hbm.at[c, pl.ds(s * RPT, RPT)])

    return _sc_prop


_sc_prop1 = _make_prop(D)


RBLK = 1024


PB = 128


def _tc1_body(x_ref, w_ref, deg_ref, g_ref, dis_ref):
    d = deg_ref[0] + deg_ref[1] + 1.0
    dis = lax.rsqrt(d)
    h = jnp.dot(x_ref[...], w_ref[...], preferred_element_type=jnp.float32,
                precision=lax.Precision.HIGHEST)
    g_ref[...] = dis * h
    dis_ref[...] = dis


_tc1 = pl.pallas_call(
    _tc1_body,
    grid=(NPAD // RBLK,),
    in_specs=[
        pl.BlockSpec((PB, 8 * D_IN), lambda i: (i, 0)),
        pl.BlockSpec((8 * D_IN, 128), lambda i: (0, 0)),
        pl.BlockSpec((NSC, PB, 128), lambda i: (0, i, 0)),
    ],
    out_specs=[
        pl.BlockSpec((PB, 128), lambda i: (i, 0)),
        pl.BlockSpec((PB, 128), lambda i: (i, 0)),
    ],
    out_shape=[
        jax.ShapeDtypeStruct((NPAD // 8, 128), jnp.float32),
        jax.ShapeDtypeStruct((NPAD // 8, 128), jnp.float32),
    ],
)


def _tc2_body(p_ref, g1_ref, dis_ref, b1_ref, w2_ref, g2_ref):
    s = (p_ref[0] + p_ref[1] + g1_ref[...]) * dis_ref[...] + b1_ref[...]
    h = jnp.maximum(s, 0.0)
    g2_ref[...] = dis_ref[...] * jnp.dot(
        h, w2_ref[...], preferred_element_type=jnp.float32,
        precision=lax.Precision.HIGHEST)


_tc2 = pl.pallas_call(
    _tc2_body,
    grid=(NPAD // RBLK,),
    in_specs=[
        pl.BlockSpec((NSC, PB, 128), lambda i: (0, i, 0)),
        pl.BlockSpec((PB, 128), lambda i: (i, 0)),
        pl.BlockSpec((PB, 128), lambda i: (i, 0)),
        pl.BlockSpec((1, 128), lambda i: (0, 0)),
        pl.BlockSpec((128, 128), lambda i: (0, 0)),
    ],
    out_specs=pl.BlockSpec((PB, 128), lambda i: (i, 0)),
    out_shape=jax.ShapeDtypeStruct((NPAD // 8, 128), jnp.float32),
)


def _tc3_body(q_ref, g2_ref, dis_ref, b2_ref, o_ref):
    o_ref[...] = ((q_ref[0] + q_ref[1] + g2_ref[...]) * dis_ref[...]
                  + b2_ref[...])


_tc3 = pl.pallas_call(
    _tc3_body,
    grid=(NPAD // RBLK,),
    in_specs=[
        pl.BlockSpec((NSC, PB, 128), lambda i: (0, i, 0)),
        pl.BlockSpec((PB, 128), lambda i: (i, 0)),
        pl.BlockSpec((PB, 128), lambda i: (i, 0)),
        pl.BlockSpec((1, 128), lambda i: (0, 0)),
    ],
    out_specs=pl.BlockSpec((PB, 128), lambda i: (i, 0)),
    out_shape=jax.ShapeDtypeStruct((NPAD // 8, 128), jnp.float32),
)


def kernel(x, edge_index, W1, b1, W2, b2):
    x_pad = jnp.pad(x, ((0, NPAD - N), (0, 0)))
    ep = jnp.pad(edge_index, ((0, 0), (0, EPAD - E)), constant_values=N)
    ep_prop = ep.reshape(2, NW, NCHUNK, CROWS)
    ep_deg = ep.reshape(2, NW, NB, BATCH)

    ones_b = jnp.ones((BATCH,), jnp.float32)
    z1 = jnp.zeros((NPAD,), jnp.float32)
    z2 = jnp.zeros((NPAD, D), jnp.float32)

    x_pk = x_pad.reshape(NPAD // 8, 8 * D_IN)
    w1bd = jnp.kron(jnp.eye(8, dtype=jnp.float32), W1)

    degw = _sc_degree(ep_deg, ones_b, z1)
    g1p, disp = _tc1(x_pk, w1bd, degw)
    p = _sc_prop1(g1p.reshape(NPAD, D), ep_prop, z2)

    w2sq = jnp.pad(W2, ((0, 0), (0, D - D_OUT)))
    w2bd = jnp.kron(jnp.eye(8, dtype=jnp.float32), w2sq)
    b1t = jnp.tile(b1, 8).reshape(1, 128)
    b2t = jnp.tile(jnp.pad(b2, (0, D - D_OUT)), 8).reshape(1, 128)

    g2p = _tc2(p.reshape(NSC, NPAD // 8, 128), g1p, disp, b1t, w2bd)
    q = _sc_prop1(g2p.reshape(NPAD, D), ep_prop, z2)
    outp = _tc3(q.reshape(NSC, NPAD // 8, 128), g2p, disp, b2t)
    return outp.reshape(NPAD, D)[:N, :D_OUT]


# speedup vs baseline: 73.3022x; 1.0145x over previous
import functools

import jax
import jax.numpy as jnp
from jax import lax
from jax.experimental import pallas as pl
from jax.experimental.pallas import tpu as pltpu
from jax.experimental.pallas import tpu_sc as plsc

N = 10000
E = 320000
D_IN = 128
D = 16
D2 = 8
D_OUT = 7

NTILE = 16
NSC = 2
NW = NTILE * NSC

NPAD = 10240
RPT = NPAD // NTILE
EPAD = 327680
BATCH = 128
CROWS = 2048
NCH0 = 6
NCH1 = 4
ET0 = NCH0 * CROWS
ET1 = NCH1 * CROWS
BASE1 = NTILE * ET0
NB0 = ET0 // BATCH
NB1 = ET1 // BATCH

_MESH = plsc.VectorSubcoreMesh(core_axis_name="c", subcore_axis_name="s")


@functools.partial(
    pl.kernel,
    out_type=jax.ShapeDtypeStruct((NSC, NPAD // 8, 128), jnp.float32),
    mesh=_MESH,
    scratch_types=[
        pltpu.VMEM((NB0, BATCH), jnp.int32),
        pltpu.VMEM((BATCH,), jnp.float32),
        pltpu.VMEM((RPT,), jnp.float32),
        pltpu.VMEM((RPT // 8, 128), jnp.float32),
        pltpu.VMEM_SHARED((NPAD,), jnp.float32),
        pltpu.SemaphoreType.DMA,
        pltpu.SemaphoreType.DMA,
    ],
    compiler_params=pltpu.CompilerParams(needs_layout_passes=False),
)
def _sc_degree(dst0_hbm, dst1_hbm, ones_hbm, z1_hbm, deg_hbm, dst_v, ones_v,
               dv_v, dw_v, deg_s, sem, ssem):
    c = lax.axis_index("c")
    s = lax.axis_index("s")

    pltpu.sync_copy(ones_hbm, ones_v)
    pltpu.sync_copy(z1_hbm.at[pl.ds(s * RPT, RPT)],
                    deg_s.at[pl.ds(s * RPT, RPT)])

    def scatter_deg(nb):
        def issue(g, carry):
            pltpu.async_copy(ones_v, deg_s.at[dst_v.at[g]], ssem, add=True)
            return carry

        lax.fori_loop(0, nb, issue, 0)

        def drain(g, carry):
            pltpu.make_async_copy(ones_v, deg_s.at[dst_v.at[0]], ssem).wait()
            return carry

        lax.fori_loop(0, nb, drain, 0)

    @pl.when(c == 0)
    def _():
        pltpu.async_copy(dst0_hbm.at[s], dst_v, sem).wait()
        plsc.subcore_barrier()
        scatter_deg(NB0)

    @pl.when(c == 1)
    def _():
        pltpu.async_copy(dst1_hbm.at[s], dst_v.at[pl.ds(0, NB1)], sem).wait()
        plsc.subcore_barrier()
        scatter_deg(NB1)

    plsc.subcore_barrier()

    pltpu.sync_copy(deg_s.at[pl.ds(s * RPT, RPT)], dv_v)

    def widen(j, carry):
        idx = jnp.broadcast_to(j, (16,)).astype(jnp.int32)
        dw_v[j // 8, pl.ds((j % 8) * 16, 16)] = plsc.load_gather(dv_v, [idx])
        return carry

    lax.fori_loop(0, RPT, widen, 0, unroll=4)
    pltpu.sync_copy(dw_v, deg_hbm.at[c, pl.ds(s * (RPT // 8), RPT // 8)])


def _make_prop(d):
    @functools.partial(
        pl.kernel,
        out_type=jax.ShapeDtypeStruct((NSC, NPAD, d), jnp.float32),
        mesh=_MESH,
        scratch_types=[
            pltpu.VMEM((NCH0, CROWS), jnp.int32),
            pltpu.VMEM((NCH0, CROWS), jnp.int32),
            pltpu.VMEM((CROWS, d), jnp.float32),
            pltpu.VMEM((CROWS, d), jnp.float32),
            pltpu.VMEM_SHARED((NPAD, d), jnp.float32),
            pltpu.VMEM_SHARED((NPAD, d), jnp.float32),
            pltpu.SemaphoreType.DMA,
            pltpu.SemaphoreType.DMA,
            pltpu.SemaphoreType.DMA,
        ],
        compiler_params=pltpu.CompilerParams(use_tc_tiling_on_sc=False),
    )
    def _sc_prop(g_hbm, src_hbm, dst_hbm, z2_hbm, p_hbm, src_v, dst_v,
                 rows0_v, rows1_v, acc_s, g_s, sem0, sem1, semg):
        c = lax.axis_index("c")
        s = lax.axis_index("s")

        cpg = pltpu.async_copy(g_hbm.at[pl.ds(s * RPT, RPT)],
                               g_s.at[pl.ds(s * RPT, RPT)], semg)
        pltpu.sync_copy(z2_hbm.at[pl.ds(s * RPT, RPT)],
                        acc_s.at[pl.ds(s * RPT, RPT)])

        rows = (rows0_v, rows1_v)
        sems = (sem0, sem1)

        def run(nch, base):
            for k in range(nch):
                pltpu.async_copy(
                    src_hbm.at[pl.ds(base + k * CROWS, CROWS)],
                    src_v.at[k], sem0)
                pltpu.async_copy(
                    dst_hbm.at[pl.ds(base + k * CROWS, CROWS)],
                    dst_v.at[k], sem1)
            for k in range(nch):
                pltpu.make_async_copy(
                    src_hbm.at[pl.ds(base, CROWS)], src_v.at[0], sem0).wait()
                pltpu.make_async_copy(
                    dst_hbm.at[pl.ds(base, CROWS)], dst_v.at[0], sem1).wait()
            cpg.wait()
            plsc.subcore_barrier()

            pltpu.async_copy(g_s.at[src_v.at[0]], rows0_v, sem0)
            for ci in range(nch):
                if ci + 1 < nch:
                    pltpu.async_copy(g_s.at[src_v.at[ci + 1]],
                                     rows[(ci + 1) % 2], sems[(ci + 1) % 2])
                pltpu.make_async_copy(g_s.at[src_v.at[ci]],
                                      rows[ci % 2], sems[ci % 2]).wait()
                pltpu.sync_copy(rows[ci % 2], acc_s.at[dst_v.at[ci]], add=True)

        @pl.when(c == 0)
        def _():
            run(NCH0, s * ET0)

        @pl.when(c == 1)
        def _():
            run(NCH1, BASE1 + s * ET1)

        plsc.subcore_barrier()

        pltpu.sync_copy(acc_s.at[pl.ds(s * RPT, RPT)],
                        p_hbm.at[c, pl.ds(s * RPT, RPT)])

    return _sc_prop


_sc_prop1 = _make_prop(D)


RBLK = 1024


PB = 128


def _tc1_body(x_ref, w_ref, deg_ref, g_ref, dis_ref):
    d = deg_ref[0] + deg_ref[1] + 1.0
    dis = lax.rsqrt(d)
    h = jnp.dot(x_ref[...], w_ref[...], preferred_element_type=jnp.float32,
                precision=lax.Precision.HIGHEST)
    g_ref[...] = dis * h
    dis_ref[...] = dis


_tc1 = pl.pallas_call(
    _tc1_body,
    grid=(NPAD // RBLK,),
    in_specs=[
        pl.BlockSpec((PB, 8 * D_IN), lambda i: (i, 0)),
        pl.BlockSpec((8 * D_IN, 128), lambda i: (0, 0)),
        pl.BlockSpec((NSC, PB, 128), lambda i: (0, i, 0)),
    ],
    out_specs=[
        pl.BlockSpec((PB, 128), lambda i: (i, 0)),
        pl.BlockSpec((PB, 128), lambda i: (i, 0)),
    ],
    out_shape=[
        jax.ShapeDtypeStruct((NPAD // 8, 128), jnp.float32),
        jax.ShapeDtypeStruct((NPAD // 8, 128), jnp.float32),
    ],
)


def _tc2_body(p_ref, g1_ref, dis_ref, b1_ref, w2_ref, g2_ref):
    s = (p_ref[0] + p_ref[1] + g1_ref[...]) * dis_ref[...] + b1_ref[...]
    h = jnp.maximum(s, 0.0)
    g2_ref[...] = dis_ref[...] * jnp.dot(
        h, w2_ref[...], preferred_element_type=jnp.float32,
        precision=lax.Precision.HIGHEST)


_tc2 = pl.pallas_call(
    _tc2_body,
    grid=(NPAD // RBLK,),
    in_specs=[
        pl.BlockSpec((NSC, PB, 128), lambda i: (0, i, 0)),
        pl.BlockSpec((PB, 128), lambda i: (i, 0)),
        pl.BlockSpec((PB, 128), lambda i: (i, 0)),
        pl.BlockSpec((1, 128), lambda i: (0, 0)),
        pl.BlockSpec((128, 128), lambda i: (0, 0)),
    ],
    out_specs=pl.BlockSpec((PB, 128), lambda i: (i, 0)),
    out_shape=jax.ShapeDtypeStruct((NPAD // 8, 128), jnp.float32),
)


def _tc3_body(q_ref, g2_ref, dis_ref, b2_ref, o_ref):
    o_ref[...] = ((q_ref[0] + q_ref[1] + g2_ref[...]) * dis_ref[...]
                  + b2_ref[...])


_tc3 = pl.pallas_call(
    _tc3_body,
    grid=(NPAD // RBLK,),
    in_specs=[
        pl.BlockSpec((NSC, PB, 128), lambda i: (0, i, 0)),
        pl.BlockSpec((PB, 128), lambda i: (i, 0)),
        pl.BlockSpec((PB, 128), lambda i: (i, 0)),
        pl.BlockSpec((1, 128), lambda i: (0, 0)),
    ],
    out_specs=pl.BlockSpec((PB, 128), lambda i: (i, 0)),
    out_shape=jax.ShapeDtypeStruct((NPAD // 8, 128), jnp.float32),
)


def kernel(x, edge_index, W1, b1, W2, b2):
    x_pad = jnp.pad(x, ((0, NPAD - N), (0, 0)))
    ep = jnp.pad(edge_index, ((0, 0), (0, EPAD - E)), constant_values=N)
    src_e, dst_e = ep[0], ep[1]
    dst_deg0 = dst_e[:BASE1].reshape(NTILE, NB0, BATCH)
    dst_deg1 = dst_e[BASE1:].reshape(NTILE, NB1, BATCH)

    ones_b = jnp.ones((BATCH,), jnp.float32)
    z1 = jnp.zeros((NPAD,), jnp.float32)
    z2 = jnp.zeros((NPAD, D), jnp.float32)

    x_pk = x_pad.reshape(NPAD // 8, 8 * D_IN)
    w1bd = jnp.kron(jnp.eye(8, dtype=jnp.float32), W1)

    degw = _sc_degree(dst_deg0, dst_deg1, ones_b, z1)
    g1p, disp = _tc1(x_pk, w1bd, degw)
    p = _sc_prop1(g1p.reshape(NPAD, D), src_e, dst_e, z2)

    w2sq = jnp.pad(W2, ((0, 0), (0, D - D_OUT)))
    w2bd = jnp.kron(jnp.eye(8, dtype=jnp.float32), w2sq)
    b1t = jnp.tile(b1, 8).reshape(1, 128)
    b2t = jnp.tile(jnp.pad(b2, (0, D - D_OUT)), 8).reshape(1, 128)

    g2p = _tc2(p.reshape(NSC, NPAD // 8, 128), g1p, disp, b1t, w2bd)
    q = _sc_prop1(g2p.reshape(NPAD, D), src_e, dst_e, z2)
    outp = _tc3(q.reshape(NSC, NPAD // 8, 128), g2p, disp, b2t)
    return outp.reshape(NPAD, D)[:N, :D_OUT]
